# baseline (speedup 1.0000x reference)
import functools

import jax
import jax.numpy as jnp
from jax import lax
from jax.experimental import pallas as pl
from jax.experimental.pallas import tpu as pltpu
from jax.experimental.pallas import tpu_sc as plsc

_B = 4096
_EMB = 64
_STATE = 64
_HID = 128
_MAX_OUT = 200
_PAD_ID = 0

_NC = 2
_NS = 16
_NW = _NC * _NS

_G = 4
_BPW = _B // _NW
_NCHUNK = _BPW // _G
_CROWS = _G * _MAX_OUT


_V = 1000400
_NBLK = _V // 128
_TAIL = _V - _NBLK * 128


def _sc_transpose(tab_t, tail_flat):
    mesh = plsc.VectorSubcoreMesh(core_axis_name="c", subcore_axis_name="s")
    nb = (_NBLK + _NW - 1) // _NW
    nb = nb + (nb % 2)

    @functools.partial(
        pl.kernel,
        mesh=mesh,
        out_type=jax.ShapeDtypeStruct((_V * _EMB,), jnp.float32),
        scratch_types=[
            pltpu.VMEM((_EMB, 128), jnp.float32),
            pltpu.VMEM((_EMB, 128), jnp.float32),
            pltpu.VMEM((128 * _EMB,), jnp.float32),
            pltpu.VMEM((128 * _EMB,), jnp.float32),
            pltpu.VMEM((_TAIL * _EMB,), jnp.float32),
            pltpu.SemaphoreType.DMA,
            pltpu.SemaphoreType.DMA,
            pltpu.SemaphoreType.DMA,
            pltpu.SemaphoreType.DMA,
        ],
        compiler_params=pltpu.CompilerParams(needs_layout_passes=False),
    )
    def k(tab_hbm, tail_hbm, out_hbm, st0, st1, rb0, rb1, tb, si0, si1,
          so0, so1):
        wid = lax.axis_index("s") * _NC + lax.axis_index("c")
        stages = (st0, st1)
        rowbufs = (rb0, rb1)
        sin = (si0, si1)
        sout = (so0, so1)
        iota64 = lax.iota(jnp.int32, 16) * _EMB

        def id0_of(i):
            bid = jnp.minimum(wid + i * _NW, _NBLK - 1)
            return pl.multiple_of(bid * 128, 128)

        def start_in(i, slot):
            pltpu.async_copy(tab_hbm.at[:, pl.ds(id0_of(i), 128)],
                             stages[slot], sin[slot])

        def wait_in(slot):
            pltpu.make_async_copy(tab_hbm.at[:, pl.ds(0, 128)],
                                  stages[slot], sin[slot]).wait()

        def start_out(i, slot):
            pltpu.async_copy(rowbufs[slot],
                             out_hbm.at[pl.ds(id0_of(i) * _EMB, 128 * _EMB)],
                             sout[slot])

        def wait_out(slot):
            pltpu.make_async_copy(rowbufs[slot],
                                  out_hbm.at[pl.ds(0, 128 * _EMB)],
                                  sout[slot]).wait()

        def compute(slot):
            stage = stages[slot]
            rb = rowbufs[slot]

            def k_body(kk, carry):
                for ig in range(8):
                    v = stage[kk, pl.ds(16 * ig, 16)]
                    plsc.store_scatter(rb, [iota64 + (ig * 16 * _EMB + kk)],
                                       v)
                return carry

            lax.fori_loop(0, _EMB, k_body, 0)

        @pl.when(wid == 0)
        def _():
            pltpu.sync_copy(tail_hbm, tb)
            pltpu.sync_copy(tb, out_hbm.at[pl.ds(_NBLK * 128 * _EMB,
                                                 _TAIL * _EMB)])

        start_in(0, 0)
        start_in(1, 1)

        def pair_body(i2, carry):
            i = i2 * 2
            for s in range(2):
                wait_in(s)

                @pl.when(i + s >= 2)
                def _():
                    wait_out(s)

                compute(s)
                start_out(i + s, s)

                @pl.when(i + s + 2 < nb)
                def _():
                    start_in(i + s + 2, s)

            return carry

        lax.fori_loop(0, nb // 2, pair_body, 0)
        wait_out(0)
        wait_out(1)

    return k(tab_t, tail_flat)


def _sc_gather(table, idx, chunk):
    n = idx.shape[0]
    per_w = n // _NW
    nch = per_w // chunk
    mesh = plsc.VectorSubcoreMesh(core_axis_name="c", subcore_axis_name="s")

    @functools.partial(
        pl.kernel,
        mesh=mesh,
        out_type=jax.ShapeDtypeStruct((n, _EMB), jnp.float32),
        scratch_types=[
            pltpu.VMEM((chunk,), jnp.int32),
            pltpu.VMEM((chunk, _EMB), jnp.float32),
            pltpu.SemaphoreType.DMA,
        ],
        compiler_params=pltpu.CompilerParams(use_tc_tiling_on_sc=False),
    )
    def k(table_hbm, idx_hbm, out_hbm, idx_v, rows_v, sem):
        wid = lax.axis_index("s") * _NC + lax.axis_index("c")
        base = wid * per_w

        def body(i, carry):
            off = base + i * chunk
            pltpu.sync_copy(idx_hbm.at[pl.ds(off, chunk)], idx_v)
            pltpu.async_copy(table_hbm.at[idx_v], rows_v, sem).wait()
            pltpu.sync_copy(rows_v, out_hbm.at[pl.ds(off, chunk)])
            return carry

        lax.fori_loop(0, nch, body, 0)

    return k(table, idx)


def _sc_scores(table, idx, mlp):
    mesh = plsc.VectorSubcoreMesh(core_axis_name="c", subcore_axis_name="s")
    njg = (_MAX_OUT + 15) // 16

    @functools.partial(
        pl.kernel,
        mesh=mesh,
        out_type=jax.ShapeDtypeStruct((_B, _MAX_OUT), jnp.float32),
        scratch_types=[
            pltpu.VMEM((_CROWS,), jnp.int32),
            pltpu.VMEM((_CROWS,), jnp.int32),
            pltpu.VMEM((_CROWS, _EMB), jnp.float32),
            pltpu.VMEM((_CROWS, _EMB), jnp.float32),
            pltpu.VMEM((_BPW, _EMB), jnp.float32),
            pltpu.VMEM((_G, _MAX_OUT), jnp.float32),
            pltpu.SemaphoreType.DMA,
            pltpu.SemaphoreType.DMA,
        ],
        compiler_params=pltpu.CompilerParams(use_tc_tiling_on_sc=False,
                                             needs_layout_passes=False),
    )
    def k(table_hbm, idx_hbm, mlp_hbm, out_hbm, idx_v0, idx_v1, rows_v0,
          rows_v1, mlp_v, scores_v, sem0, sem1):
        wid = lax.axis_index("s") * _NC + lax.axis_index("c")
        b0 = wid * _BPW
        pltpu.sync_copy(mlp_hbm.at[pl.ds(b0, _BPW)], mlp_v)

        iota16 = lax.iota(jnp.int32, 16)
        sh_idx = [jnp.bitwise_xor(iota16, d) for d in (8, 4, 2, 1)]
        idx_bufs = (idx_v0, idx_v1)
        row_bufs = (rows_v0, rows_v1)
        sems = (sem0, sem1)

        def start(ci, slot):
            pair0 = (b0 + ci * _G) * _MAX_OUT
            pltpu.sync_copy(idx_hbm.at[pl.ds(pair0, _CROWS)],
                            idx_bufs[slot])
            pltpu.async_copy(table_hbm.at[idx_bufs[slot]], row_bufs[slot],
                             sems[slot])

        def compute(ci, slot):
            rows_v = row_bufs[slot]

            def b_body(bb, carry2):
                b_loc = ci * _G + bb
                m = [plsc.load_gather(mlp_v, [jnp.full((16,), b_loc,
                                                       jnp.int32),
                                              iota16 + 16 * kk])
                     for kk in range(4)]

                def jg_body(jg, carry3):
                    acc = jnp.zeros((16,), jnp.float32)
                    for l in range(16):
                        row = jnp.minimum(bb * _MAX_OUT + jg * 16 + l,
                                          _CROWS - 1)
                        rv = jnp.full((16,), row, jnp.int32)
                        prod = (
                            plsc.load_gather(rows_v, [rv, iota16]) * m[0]
                            + plsc.load_gather(rows_v, [rv, iota16 + 16])
                            * m[1]
                            + plsc.load_gather(rows_v, [rv, iota16 + 32])
                            * m[2]
                            + plsc.load_gather(rows_v, [rv, iota16 + 48])
                            * m[3])
                        for si in sh_idx:
                            prod = prod + prod.at[si].get(
                                mode="promise_in_bounds")
                        acc = jnp.where(iota16 == l, prod, acc)
                    col = iota16 + jg * 16
                    plsc.store_scatter(
                        scores_v,
                        [jnp.full((16,), bb, jnp.int32), col],
                        acc, mask=col < _MAX_OUT)
                    return carry3

                lax.fori_loop(0, njg, jg_body, 0)
                return carry2

            lax.fori_loop(0, _G, b_body, 0)
            pltpu.sync_copy(scores_v, out_hbm.at[pl.ds(b0 + ci * _G, _G)])

        def wait(slot):
            pltpu.make_async_copy(table_hbm.at[idx_bufs[slot]],
                                  row_bufs[slot], sems[slot]).wait()

        start(0, 0)

        def pair_body(i, carry):
            ci = i * 2
            wait(0)
            start(ci + 1, 1)
            compute(ci, 0)
            wait(1)

            @pl.when(ci + 2 < _NCHUNK)
            def _():
                start(ci + 2, 0)

            compute(ci + 1, 1)
            return carry

        lax.fori_loop(0, _NCHUNK // 2, pair_body, 0)

    return k(table, idx, mlp)


def _dense_body(x_ref, h_ref, c_ref, q_ref,
                wx0, wx1, wx2, wx3, wh0, wh1, wh2, wh3,
                bih, bhh, w1a, w1b, b1, w2, b2,
                h_out, c_out, mlp_out):
    x = x_ref[...]
    h = h_ref[...]
    c = c_ref[...]
    q = q_ref[...]
    b4 = bih[...] + bhh[...]

    def gate(wx, wh, k):
        return (jnp.dot(x, wx[...], preferred_element_type=jnp.float32)
                + jnp.dot(h, wh[...], preferred_element_type=jnp.float32)
                + b4[k:k + 1, :])

    gi = jax.nn.sigmoid(gate(wx0, wh0, 0))
    gf = jax.nn.sigmoid(gate(wx1, wh1, 1))
    gg = jnp.tanh(gate(wx2, wh2, 2))
    go = jax.nn.sigmoid(gate(wx3, wh3, 3))
    c_new = gf * c + gi * gg
    h_new = go * jnp.tanh(c_new)

    hidden = jax.nn.relu(
        jnp.dot(h_new, w1a[...], preferred_element_type=jnp.float32)
        + jnp.dot(q, w1b[...], preferred_element_type=jnp.float32)
        + b1[...])
    mlp = jax.nn.relu(
        jnp.dot(hidden, w2[...], preferred_element_type=jnp.float32)
        + b2[...])

    h_out[...] = h_new
    c_out[...] = c_new
    mlp_out[...] = mlp


def _finish_body(scores_ref, rel_ref, ent_ref, noise_ref,
                 logits_ref, aid_ref, loss_ref, chosen_ref, nexte_ref):
    rel = rel_ref[...]
    ent = ent_ref[...]
    noise = noise_ref[...]

    scores = jnp.where(rel == _PAD_ID, jnp.float32(-99999.0),
                       scores_ref[...])

    m = jnp.max(scores, axis=-1, keepdims=True)
    shifted = scores - m
    logits = shifted - jnp.log(jnp.sum(jnp.exp(shifted), axis=-1,
                                       keepdims=True))

    z = logits + noise
    zmax = jnp.max(z, axis=-1, keepdims=True)
    iota = lax.broadcasted_iota(jnp.int32, z.shape, 1)
    aid = jnp.min(jnp.where(z == zmax, iota, jnp.int32(_MAX_OUT)), axis=-1,
                  keepdims=True)

    sel = iota == aid
    loss = -jnp.sum(jnp.where(sel, logits, jnp.float32(0.0)), axis=-1,
                    keepdims=True)
    chosen = jnp.sum(jnp.where(sel, rel, jnp.int32(0)), axis=-1,
                     keepdims=True)
    nexte = jnp.sum(jnp.where(sel, ent, jnp.int32(0)), axis=-1,
                    keepdims=True)

    logits_ref[...] = logits
    aid_ref[...] = aid
    loss_ref[...] = loss
    chosen_ref[...] = chosen
    nexte_ref[...] = nexte


def kernel(prev_state_h, prev_state_c, prev_relation, current_entity,
           actions_id, queries, table, W_ih, W_hh, b_ih, b_hh, W1, b1, W2, b2):
    del current_entity

    rel2d = actions_id[:, :, 0]
    ent2d = actions_id[:, :, 1]

    tail_flat = table[_NBLK * 128:].reshape(-1)
    tab_lin = _sc_transpose(table.T, tail_flat).reshape(_V, _EMB)

    small_idx = jnp.concatenate(
        [prev_relation.astype(jnp.int32), queries.astype(jnp.int32)])
    small_rows = _sc_gather(tab_lin, small_idx, chunk=256)
    prev_emb = small_rows[:_B]
    q_emb = small_rows[_B:]

    wiht = W_ih.T
    whht = W_hh.T
    wx = [wiht[:, k * _STATE:(k + 1) * _STATE] for k in range(4)]
    wh = [whht[:, k * _STATE:(k + 1) * _STATE] for k in range(4)]
    bih4 = b_ih.reshape(4, _STATE)
    bhh4 = b_hh.reshape(4, _STATE)
    w1t = W1.T
    w1a = w1t[:_STATE]
    w1b = w1t[_STATE:]
    b1r = b1.reshape(1, _HID)
    w2t = W2.T
    b2r = b2.reshape(1, _EMB)

    bm = 512
    grid = _B // bm
    row_spec = pl.BlockSpec((bm, _EMB), lambda i: (i, 0))
    full = lambda shape: pl.BlockSpec(shape, lambda i: tuple(0 for _ in shape))
    h_new, c_new, mlp = pl.pallas_call(
        _dense_body,
        grid=(grid,),
        in_specs=[row_spec, row_spec, row_spec, row_spec]
        + [full((_EMB, _STATE))] * 8
        + [full((4, _STATE))] * 2
        + [full((_STATE, _HID)), full((_EMB, _HID)), full((1, _HID)),
           full((_HID, _EMB)), full((1, _EMB))],
        out_specs=[row_spec, row_spec, row_spec],
        out_shape=[jax.ShapeDtypeStruct((_B, _STATE), jnp.float32)] * 3,
    )(prev_emb, prev_state_h, prev_state_c, q_emb,
      *wx, *wh, bih4, bhh4, w1a, w1b, b1r, w2t, b2r)

    scores = _sc_scores(tab_lin, rel2d.reshape(-1), mlp)

    noise = jax.random.gumbel(jax.random.key(42), (_B, _MAX_OUT), jnp.float32)

    bm2 = 512
    grid2 = _B // bm2
    spec2d = pl.BlockSpec((bm2, _MAX_OUT), lambda i: (i, 0))
    spec1 = pl.BlockSpec((bm2, 1), lambda i: (i, 0))
    logits, aid, loss, chosen, nexte = pl.pallas_call(
        _finish_body,
        grid=(grid2,),
        in_specs=[spec2d, spec2d, spec2d, spec2d],
        out_specs=[spec2d, spec1, spec1, spec1, spec1],
        out_shape=[
            jax.ShapeDtypeStruct((_B, _MAX_OUT), jnp.float32),
            jax.ShapeDtypeStruct((_B, 1), jnp.int32),
            jax.ShapeDtypeStruct((_B, 1), jnp.float32),
            jax.ShapeDtypeStruct((_B, 1), jnp.int32),
            jax.ShapeDtypeStruct((_B, 1), jnp.int32),
        ],
    )(scores, rel2d, ent2d, noise)

    return (loss.reshape(_B), h_new, c_new, logits,
            aid.reshape(_B), nexte.reshape(_B), chosen.reshape(_B))


# speedup vs baseline: 1.8698x; 1.8698x over previous
import functools

import jax
import jax.numpy as jnp
from jax import lax
from jax.experimental import pallas as pl
from jax.experimental.pallas import tpu as pltpu
from jax.experimental.pallas import tpu_sc as plsc

_B = 4096
_EMB = 64
_STATE = 64
_HID = 128
_MAX_OUT = 200
_PAD_ID = 0

_NC = 2
_NS = 16
_NW = _NC * _NS

_G = 4
_BPW = _B // _NW
_NCHUNK = _BPW // _G
_CROWS = _G * _MAX_OUT


_V = 1000400
_NBLK = _V // 128
_TAIL = _V - _NBLK * 128


def _sc_transpose(tab_t, tail_flat):
    mesh = plsc.VectorSubcoreMesh(core_axis_name="c", subcore_axis_name="s")
    nb = (_NBLK + _NW - 1) // _NW
    nb = nb + (nb % 2)

    @functools.partial(
        pl.kernel,
        mesh=mesh,
        out_type=jax.ShapeDtypeStruct((_V * _EMB,), jnp.float32),
        scratch_types=[
            pltpu.VMEM((_EMB, 128), jnp.float32),
            pltpu.VMEM((_EMB, 128), jnp.float32),
            pltpu.VMEM((128 * _EMB,), jnp.float32),
            pltpu.VMEM((128 * _EMB,), jnp.float32),
            pltpu.VMEM((_TAIL * _EMB,), jnp.float32),
            pltpu.SemaphoreType.DMA,
            pltpu.SemaphoreType.DMA,
            pltpu.SemaphoreType.DMA,
            pltpu.SemaphoreType.DMA,
        ],
        compiler_params=pltpu.CompilerParams(needs_layout_passes=False),
    )
    def k(tab_hbm, tail_hbm, out_hbm, st0, st1, rb0, rb1, tb, si0, si1,
          so0, so1):
        wid = lax.axis_index("s") * _NC + lax.axis_index("c")
        stages = (st0, st1)
        rowbufs = (rb0, rb1)
        sin = (si0, si1)
        sout = (so0, so1)
        iota64 = lax.iota(jnp.int32, 16) * _EMB

        def id0_of(i):
            bid = jnp.minimum(wid + i * _NW, _NBLK - 1)
            return pl.multiple_of(bid * 128, 128)

        def start_in(i, slot):
            pltpu.async_copy(tab_hbm.at[:, pl.ds(id0_of(i), 128)],
                             stages[slot], sin[slot])

        def wait_in(slot):
            pltpu.make_async_copy(tab_hbm.at[:, pl.ds(0, 128)],
                                  stages[slot], sin[slot]).wait()

        def start_out(i, slot):
            pltpu.async_copy(rowbufs[slot],
                             out_hbm.at[pl.ds(id0_of(i) * _EMB, 128 * _EMB)],
                             sout[slot])

        def wait_out(slot):
            pltpu.make_async_copy(rowbufs[slot],
                                  out_hbm.at[pl.ds(0, 128 * _EMB)],
                                  sout[slot]).wait()

        def compute(slot):
            stage = stages[slot]
            rb = rowbufs[slot]
            iota16 = lax.iota(jnp.int32, 16)

            def c_body(c0, carry):
                kvec = jnp.bitwise_and(c0 + iota16, _EMB - 1)
                for ig in range(8):
                    idv = iota16 + 16 * ig
                    v = plsc.load_gather(stage, [kvec, idv])
                    plsc.store_scatter(rb, [idv * _EMB + kvec], v)
                return carry

            lax.fori_loop(0, _EMB, c_body, 0)

        @pl.when(wid == 0)
        def _():
            pltpu.sync_copy(tail_hbm, tb)
            pltpu.sync_copy(tb, out_hbm.at[pl.ds(_NBLK * 128 * _EMB,
                                                 _TAIL * _EMB)])

        start_in(0, 0)
        start_in(1, 1)

        def pair_body(i2, carry):
            i = i2 * 2
            for s in range(2):
                wait_in(s)

                @pl.when(i + s >= 2)
                def _():
                    wait_out(s)

                compute(s)
                start_out(i + s, s)

                @pl.when(i + s + 2 < nb)
                def _():
                    start_in(i + s + 2, s)

            return carry

        lax.fori_loop(0, nb // 2, pair_body, 0)
        wait_out(0)
        wait_out(1)

    return k(tab_t, tail_flat)


def _sc_gather(table, idx, chunk):
    n = idx.shape[0]
    per_w = n // _NW
    nch = per_w // chunk
    mesh = plsc.VectorSubcoreMesh(core_axis_name="c", subcore_axis_name="s")

    @functools.partial(
        pl.kernel,
        mesh=mesh,
        out_type=jax.ShapeDtypeStruct((n, _EMB), jnp.float32),
        scratch_types=[
            pltpu.VMEM((chunk,), jnp.int32),
            pltpu.VMEM((chunk, _EMB), jnp.float32),
            pltpu.SemaphoreType.DMA,
        ],
        compiler_params=pltpu.CompilerParams(use_tc_tiling_on_sc=False),
    )
    def k(table_hbm, idx_hbm, out_hbm, idx_v, rows_v, sem):
        wid = lax.axis_index("s") * _NC + lax.axis_index("c")
        base = wid * per_w

        def body(i, carry):
            off = base + i * chunk
            pltpu.sync_copy(idx_hbm.at[pl.ds(off, chunk)], idx_v)
            pltpu.async_copy(table_hbm.at[idx_v], rows_v, sem).wait()
            pltpu.sync_copy(rows_v, out_hbm.at[pl.ds(off, chunk)])
            return carry

        lax.fori_loop(0, nch, body, 0)

    return k(table, idx)


def _sc_scores(table, idx, mlp):
    mesh = plsc.VectorSubcoreMesh(core_axis_name="c", subcore_axis_name="s")
    njg = (_MAX_OUT + 15) // 16

    @functools.partial(
        pl.kernel,
        mesh=mesh,
        out_type=jax.ShapeDtypeStruct((_B, _MAX_OUT), jnp.float32),
        scratch_types=[
            pltpu.VMEM((_CROWS,), jnp.int32),
            pltpu.VMEM((_CROWS,), jnp.int32),
            pltpu.VMEM((_CROWS, _EMB), jnp.float32),
            pltpu.VMEM((_CROWS, _EMB), jnp.float32),
            pltpu.VMEM((_BPW, _EMB), jnp.float32),
            pltpu.VMEM((_G, _MAX_OUT), jnp.float32),
            pltpu.SemaphoreType.DMA,
            pltpu.SemaphoreType.DMA,
        ],
        compiler_params=pltpu.CompilerParams(use_tc_tiling_on_sc=False,
                                             needs_layout_passes=False),
    )
    def k(table_hbm, idx_hbm, mlp_hbm, out_hbm, idx_v0, idx_v1, rows_v0,
          rows_v1, mlp_v, scores_v, sem0, sem1):
        wid = lax.axis_index("s") * _NC + lax.axis_index("c")
        b0 = wid * _BPW
        pltpu.sync_copy(mlp_hbm.at[pl.ds(b0, _BPW)], mlp_v)

        iota16 = lax.iota(jnp.int32, 16)
        sh_idx = [jnp.bitwise_xor(iota16, d) for d in (8, 4, 2, 1)]
        idx_bufs = (idx_v0, idx_v1)
        row_bufs = (rows_v0, rows_v1)
        sems = (sem0, sem1)

        def start(ci, slot):
            pair0 = (b0 + ci * _G) * _MAX_OUT
            pltpu.sync_copy(idx_hbm.at[pl.ds(pair0, _CROWS)],
                            idx_bufs[slot])
            pltpu.async_copy(table_hbm.at[idx_bufs[slot]], row_bufs[slot],
                             sems[slot])

        def compute(ci, slot):
            rows_v = row_bufs[slot]

            def b_body(bb, carry2):
                b_loc = ci * _G + bb
                m = [plsc.load_gather(mlp_v, [jnp.full((16,), b_loc,
                                                       jnp.int32),
                                              iota16 + 16 * kk])
                     for kk in range(4)]

                def jg_body(jg, carry3):
                    acc = jnp.zeros((16,), jnp.float32)
                    for l in range(16):
                        row = jnp.minimum(bb * _MAX_OUT + jg * 16 + l,
                                          _CROWS - 1)
                        rv = jnp.full((16,), row, jnp.int32)
                        prod = (
                            plsc.load_gather(rows_v, [rv, iota16]) * m[0]
                            + plsc.load_gather(rows_v, [rv, iota16 + 16])
                            * m[1]
                            + plsc.load_gather(rows_v, [rv, iota16 + 32])
                            * m[2]
                            + plsc.load_gather(rows_v, [rv, iota16 + 48])
                            * m[3])
                        for si in sh_idx:
                            prod = prod + prod.at[si].get(
                                mode="promise_in_bounds")
                        acc = jnp.where(iota16 == l, prod, acc)
                    col = iota16 + jg * 16
                    plsc.store_scatter(
                        scores_v,
                        [jnp.full((16,), bb, jnp.int32), col],
                        acc, mask=col < _MAX_OUT)
                    return carry3

                lax.fori_loop(0, njg, jg_body, 0)
                return carry2

            lax.fori_loop(0, _G, b_body, 0)
            pltpu.sync_copy(scores_v, out_hbm.at[pl.ds(b0 + ci * _G, _G)])

        def wait(slot):
            pltpu.make_async_copy(table_hbm.at[idx_bufs[slot]],
                                  row_bufs[slot], sems[slot]).wait()

        start(0, 0)

        def pair_body(i, carry):
            ci = i * 2
            wait(0)
            start(ci + 1, 1)
            compute(ci, 0)
            wait(1)

            @pl.when(ci + 2 < _NCHUNK)
            def _():
                start(ci + 2, 0)

            compute(ci + 1, 1)
            return carry

        lax.fori_loop(0, _NCHUNK // 2, pair_body, 0)

    return k(table, idx, mlp)


def _dense_body(x_ref, h_ref, c_ref, q_ref,
                wx0, wx1, wx2, wx3, wh0, wh1, wh2, wh3,
                bih, bhh, w1a, w1b, b1, w2, b2,
                h_out, c_out, mlp_out):
    x = x_ref[...]
    h = h_ref[...]
    c = c_ref[...]
    q = q_ref[...]
    b4 = bih[...] + bhh[...]

    def gate(wx, wh, k):
        return (jnp.dot(x, wx[...], preferred_element_type=jnp.float32)
                + jnp.dot(h, wh[...], preferred_element_type=jnp.float32)
                + b4[k:k + 1, :])

    gi = jax.nn.sigmoid(gate(wx0, wh0, 0))
    gf = jax.nn.sigmoid(gate(wx1, wh1, 1))
    gg = jnp.tanh(gate(wx2, wh2, 2))
    go = jax.nn.sigmoid(gate(wx3, wh3, 3))
    c_new = gf * c + gi * gg
    h_new = go * jnp.tanh(c_new)

    hidden = jax.nn.relu(
        jnp.dot(h_new, w1a[...], preferred_element_type=jnp.float32)
        + jnp.dot(q, w1b[...], preferred_element_type=jnp.float32)
        + b1[...])
    mlp = jax.nn.relu(
        jnp.dot(hidden, w2[...], preferred_element_type=jnp.float32)
        + b2[...])

    h_out[...] = h_new
    c_out[...] = c_new
    mlp_out[...] = mlp


def _finish_body(scores_ref, rel_ref, ent_ref, noise_ref,
                 logits_ref, aid_ref, loss_ref, chosen_ref, nexte_ref):
    rel = rel_ref[...]
    ent = ent_ref[...]
    noise = noise_ref[...]

    scores = jnp.where(rel == _PAD_ID, jnp.float32(-99999.0),
                       scores_ref[...])

    m = jnp.max(scores, axis=-1, keepdims=True)
    shifted = scores - m
    logits = shifted - jnp.log(jnp.sum(jnp.exp(shifted), axis=-1,
                                       keepdims=True))

    z = logits + noise
    zmax = jnp.max(z, axis=-1, keepdims=True)
    iota = lax.broadcasted_iota(jnp.int32, z.shape, 1)
    aid = jnp.min(jnp.where(z == zmax, iota, jnp.int32(_MAX_OUT)), axis=-1,
                  keepdims=True)

    sel = iota == aid
    loss = -jnp.sum(jnp.where(sel, logits, jnp.float32(0.0)), axis=-1,
                    keepdims=True)
    chosen = jnp.sum(jnp.where(sel, rel, jnp.int32(0)), axis=-1,
                     keepdims=True)
    nexte = jnp.sum(jnp.where(sel, ent, jnp.int32(0)), axis=-1,
                    keepdims=True)

    logits_ref[...] = logits
    aid_ref[...] = aid
    loss_ref[...] = loss
    chosen_ref[...] = chosen
    nexte_ref[...] = nexte


def kernel(prev_state_h, prev_state_c, prev_relation, current_entity,
           actions_id, queries, table, W_ih, W_hh, b_ih, b_hh, W1, b1, W2, b2):
    del current_entity

    rel2d = actions_id[:, :, 0]
    ent2d = actions_id[:, :, 1]

    tail_flat = table[_NBLK * 128:].reshape(-1)
    tab_lin = _sc_transpose(table.T, tail_flat).reshape(_V, _EMB)

    small_idx = jnp.concatenate(
        [prev_relation.astype(jnp.int32), queries.astype(jnp.int32)])
    small_rows = _sc_gather(tab_lin, small_idx, chunk=256)
    prev_emb = small_rows[:_B]
    q_emb = small_rows[_B:]

    wiht = W_ih.T
    whht = W_hh.T
    wx = [wiht[:, k * _STATE:(k + 1) * _STATE] for k in range(4)]
    wh = [whht[:, k * _STATE:(k + 1) * _STATE] for k in range(4)]
    bih4 = b_ih.reshape(4, _STATE)
    bhh4 = b_hh.reshape(4, _STATE)
    w1t = W1.T
    w1a = w1t[:_STATE]
    w1b = w1t[_STATE:]
    b1r = b1.reshape(1, _HID)
    w2t = W2.T
    b2r = b2.reshape(1, _EMB)

    bm = 512
    grid = _B // bm
    row_spec = pl.BlockSpec((bm, _EMB), lambda i: (i, 0))
    full = lambda shape: pl.BlockSpec(shape, lambda i: tuple(0 for _ in shape))
    h_new, c_new, mlp = pl.pallas_call(
        _dense_body,
        grid=(grid,),
        in_specs=[row_spec, row_spec, row_spec, row_spec]
        + [full((_EMB, _STATE))] * 8
        + [full((4, _STATE))] * 2
        + [full((_STATE, _HID)), full((_EMB, _HID)), full((1, _HID)),
           full((_HID, _EMB)), full((1, _EMB))],
        out_specs=[row_spec, row_spec, row_spec],
        out_shape=[jax.ShapeDtypeStruct((_B, _STATE), jnp.float32)] * 3,
    )(prev_emb, prev_state_h, prev_state_c, q_emb,
      *wx, *wh, bih4, bhh4, w1a, w1b, b1r, w2t, b2r)

    scores = _sc_scores(tab_lin, rel2d.reshape(-1), mlp)

    noise = jax.random.gumbel(jax.random.key(42), (_B, _MAX_OUT), jnp.float32)

    bm2 = 512
    grid2 = _B // bm2
    spec2d = pl.BlockSpec((bm2, _MAX_OUT), lambda i: (i, 0))
    spec1 = pl.BlockSpec((bm2, 1), lambda i: (i, 0))
    logits, aid, loss, chosen, nexte = pl.pallas_call(
        _finish_body,
        grid=(grid2,),
        in_specs=[spec2d, spec2d, spec2d, spec2d],
        out_specs=[spec2d, spec1, spec1, spec1, spec1],
        out_shape=[
            jax.ShapeDtypeStruct((_B, _MAX_OUT), jnp.float32),
            jax.ShapeDtypeStruct((_B, 1), jnp.int32),
            jax.ShapeDtypeStruct((_B, 1), jnp.float32),
            jax.ShapeDtypeStruct((_B, 1), jnp.int32),
            jax.ShapeDtypeStruct((_B, 1), jnp.int32),
        ],
    )(scores, rel2d, ent2d, noise)

    return (loss.reshape(_B), h_new, c_new, logits,
            aid.reshape(_B), nexte.reshape(_B), chosen.reshape(_B))


# speedup vs baseline: 1.9139x; 1.0235x over previous
import functools

import jax
import jax.numpy as jnp
from jax import lax
from jax.experimental import pallas as pl
from jax.experimental.pallas import tpu as pltpu
from jax.experimental.pallas import tpu_sc as plsc

_B = 4096
_EMB = 64
_STATE = 64
_HID = 128
_MAX_OUT = 200
_PAD_ID = 0

_NC = 2
_NS = 16
_NW = _NC * _NS

_G = 4
_BPW = _B // _NW
_NCHUNK = _BPW // _G
_CROWS = _G * _MAX_OUT


_V = 1000400
_NBLK = _V // 128
_TAIL = _V - _NBLK * 128


def _sc_transpose(tab_t, tail_flat):
    mesh = plsc.VectorSubcoreMesh(core_axis_name="c", subcore_axis_name="s")
    nb = (_NBLK + _NW - 1) // _NW
    nb = nb + (nb % 2)

    @functools.partial(
        pl.kernel,
        mesh=mesh,
        out_type=jax.ShapeDtypeStruct((_V * _EMB,), jnp.float32),
        scratch_types=[
            pltpu.VMEM((_EMB, 128), jnp.float32),
            pltpu.VMEM((_EMB, 128), jnp.float32),
            pltpu.VMEM((128 * _EMB,), jnp.float32),
            pltpu.VMEM((128 * _EMB,), jnp.float32),
            pltpu.VMEM((_TAIL * _EMB,), jnp.float32),
            pltpu.SemaphoreType.DMA,
            pltpu.SemaphoreType.DMA,
            pltpu.SemaphoreType.DMA,
            pltpu.SemaphoreType.DMA,
        ],
        compiler_params=pltpu.CompilerParams(needs_layout_passes=False),
    )
    def k(tab_hbm, tail_hbm, out_hbm, st0, st1, rb0, rb1, tb, si0, si1,
          so0, so1):
        wid = lax.axis_index("s") * _NC + lax.axis_index("c")
        stages = (st0, st1)
        rowbufs = (rb0, rb1)
        sin = (si0, si1)
        sout = (so0, so1)
        iota64 = lax.iota(jnp.int32, 16) * _EMB

        def id0_of(i):
            bid = jnp.minimum(wid + i * _NW, _NBLK - 1)
            return pl.multiple_of(bid * 128, 128)

        def start_in(i, slot):
            pltpu.async_copy(tab_hbm.at[:, pl.ds(id0_of(i), 128)],
                             stages[slot], sin[slot])

        def wait_in(slot):
            pltpu.make_async_copy(tab_hbm.at[:, pl.ds(0, 128)],
                                  stages[slot], sin[slot]).wait()

        def start_out(i, slot):
            pltpu.async_copy(rowbufs[slot],
                             out_hbm.at[pl.ds(id0_of(i) * _EMB, 128 * _EMB)],
                             sout[slot])

        def wait_out(slot):
            pltpu.make_async_copy(rowbufs[slot],
                                  out_hbm.at[pl.ds(0, 128 * _EMB)],
                                  sout[slot]).wait()

        def compute(slot):
            stage = stages[slot]
            rb = rowbufs[slot]
            iota16 = lax.iota(jnp.int32, 16)

            def c_body(cq, carry):
                for dc in range(4):
                    kvec = jnp.bitwise_and(cq * 4 + dc + iota16, _EMB - 1)
                    for ig in range(8):
                        idv = iota16 + 16 * ig
                        v = plsc.load_gather(stage, [kvec, idv])
                        plsc.store_scatter(rb, [idv * _EMB + kvec], v)
                return carry

            lax.fori_loop(0, _EMB // 4, c_body, 0)

        @pl.when(wid == 0)
        def _():
            pltpu.sync_copy(tail_hbm, tb)
            pltpu.sync_copy(tb, out_hbm.at[pl.ds(_NBLK * 128 * _EMB,
                                                 _TAIL * _EMB)])

        start_in(0, 0)
        start_in(1, 1)

        def pair_body(i2, carry):
            i = i2 * 2
            for s in range(2):
                wait_in(s)

                @pl.when(i + s >= 2)
                def _():
                    wait_out(s)

                compute(s)
                start_out(i + s, s)

                @pl.when(i + s + 2 < nb)
                def _():
                    start_in(i + s + 2, s)

            return carry

        lax.fori_loop(0, nb // 2, pair_body, 0)
        wait_out(0)
        wait_out(1)

    return k(tab_t, tail_flat)


def _sc_gather(table, idx, chunk):
    n = idx.shape[0]
    per_w = n // _NW
    nch = per_w // chunk
    mesh = plsc.VectorSubcoreMesh(core_axis_name="c", subcore_axis_name="s")

    @functools.partial(
        pl.kernel,
        mesh=mesh,
        out_type=jax.ShapeDtypeStruct((n, _EMB), jnp.float32),
        scratch_types=[
            pltpu.VMEM((chunk,), jnp.int32),
            pltpu.VMEM((chunk, _EMB), jnp.float32),
            pltpu.SemaphoreType.DMA,
        ],
        compiler_params=pltpu.CompilerParams(use_tc_tiling_on_sc=False),
    )
    def k(table_hbm, idx_hbm, out_hbm, idx_v, rows_v, sem):
        wid = lax.axis_index("s") * _NC + lax.axis_index("c")
        base = wid * per_w

        def body(i, carry):
            off = base + i * chunk
            pltpu.sync_copy(idx_hbm.at[pl.ds(off, chunk)], idx_v)
            pltpu.async_copy(table_hbm.at[idx_v], rows_v, sem).wait()
            pltpu.sync_copy(rows_v, out_hbm.at[pl.ds(off, chunk)])
            return carry

        lax.fori_loop(0, nch, body, 0)

    return k(table, idx)


def _sc_scores(table, idx, mlp):
    mesh = plsc.VectorSubcoreMesh(core_axis_name="c", subcore_axis_name="s")
    njg = (_MAX_OUT + 15) // 16

    @functools.partial(
        pl.kernel,
        mesh=mesh,
        out_type=jax.ShapeDtypeStruct((_B, _MAX_OUT), jnp.float32),
        scratch_types=[
            pltpu.VMEM((_CROWS,), jnp.int32),
            pltpu.VMEM((_CROWS,), jnp.int32),
            pltpu.VMEM((_CROWS, _EMB), jnp.float32),
            pltpu.VMEM((_CROWS, _EMB), jnp.float32),
            pltpu.VMEM((_BPW, _EMB), jnp.float32),
            pltpu.VMEM((_G, _MAX_OUT), jnp.float32),
            pltpu.SemaphoreType.DMA,
            pltpu.SemaphoreType.DMA,
        ],
        compiler_params=pltpu.CompilerParams(use_tc_tiling_on_sc=False,
                                             needs_layout_passes=False),
    )
    def k(table_hbm, idx_hbm, mlp_hbm, out_hbm, idx_v0, idx_v1, rows_v0,
          rows_v1, mlp_v, scores_v, sem0, sem1):
        wid = lax.axis_index("s") * _NC + lax.axis_index("c")
        b0 = wid * _BPW
        pltpu.sync_copy(mlp_hbm.at[pl.ds(b0, _BPW)], mlp_v)

        iota16 = lax.iota(jnp.int32, 16)
        sh_idx = [jnp.bitwise_xor(iota16, d) for d in (8, 4, 2, 1)]
        idx_bufs = (idx_v0, idx_v1)
        row_bufs = (rows_v0, rows_v1)
        sems = (sem0, sem1)

        def start(ci, slot):
            pair0 = (b0 + ci * _G) * _MAX_OUT
            pltpu.sync_copy(idx_hbm.at[pl.ds(pair0, _CROWS)],
                            idx_bufs[slot])
            pltpu.async_copy(table_hbm.at[idx_bufs[slot]], row_bufs[slot],
                             sems[slot])

        def compute(ci, slot):
            rows_v = row_bufs[slot]

            def b_body(bb, carry2):
                b_loc = ci * _G + bb
                m = [plsc.load_gather(mlp_v, [jnp.full((16,), b_loc,
                                                       jnp.int32),
                                              iota16 + 16 * kk])
                     for kk in range(4)]

                def jg_body(jg, carry3):
                    acc = jnp.zeros((16,), jnp.float32)
                    for l in range(16):
                        row = jnp.minimum(bb * _MAX_OUT + jg * 16 + l,
                                          _CROWS - 1)
                        rv = jnp.full((16,), row, jnp.int32)
                        prod = (
                            plsc.load_gather(rows_v, [rv, iota16]) * m[0]
                            + plsc.load_gather(rows_v, [rv, iota16 + 16])
                            * m[1]
                            + plsc.load_gather(rows_v, [rv, iota16 + 32])
                            * m[2]
                            + plsc.load_gather(rows_v, [rv, iota16 + 48])
                            * m[3])
                        for si in sh_idx:
                            prod = prod + prod.at[si].get(
                                mode="promise_in_bounds")
                        acc = jnp.where(iota16 == l, prod, acc)
                    col = iota16 + jg * 16
                    plsc.store_scatter(
                        scores_v,
                        [jnp.full((16,), bb, jnp.int32), col],
                        acc, mask=col < _MAX_OUT)
                    return carry3

                lax.fori_loop(0, njg, jg_body, 0)
                return carry2

            lax.fori_loop(0, _G, b_body, 0)
            pltpu.sync_copy(scores_v, out_hbm.at[pl.ds(b0 + ci * _G, _G)])

        def wait(slot):
            pltpu.make_async_copy(table_hbm.at[idx_bufs[slot]],
                                  row_bufs[slot], sems[slot]).wait()

        start(0, 0)

        def pair_body(i, carry):
            ci = i * 2
            wait(0)
            start(ci + 1, 1)
            compute(ci, 0)
            wait(1)

            @pl.when(ci + 2 < _NCHUNK)
            def _():
                start(ci + 2, 0)

            compute(ci + 1, 1)
            return carry

        lax.fori_loop(0, _NCHUNK // 2, pair_body, 0)

    return k(table, idx, mlp)


def _dense_body(x_ref, h_ref, c_ref, q_ref,
                wx0, wx1, wx2, wx3, wh0, wh1, wh2, wh3,
                bih, bhh, w1a, w1b, b1, w2, b2,
                h_out, c_out, mlp_out):
    x = x_ref[...]
    h = h_ref[...]
    c = c_ref[...]
    q = q_ref[...]
    b4 = bih[...] + bhh[...]

    def gate(wx, wh, k):
        return (jnp.dot(x, wx[...], preferred_element_type=jnp.float32)
                + jnp.dot(h, wh[...], preferred_element_type=jnp.float32)
                + b4[k:k + 1, :])

    gi = jax.nn.sigmoid(gate(wx0, wh0, 0))
    gf = jax.nn.sigmoid(gate(wx1, wh1, 1))
    gg = jnp.tanh(gate(wx2, wh2, 2))
    go = jax.nn.sigmoid(gate(wx3, wh3, 3))
    c_new = gf * c + gi * gg
    h_new = go * jnp.tanh(c_new)

    hidden = jax.nn.relu(
        jnp.dot(h_new, w1a[...], preferred_element_type=jnp.float32)
        + jnp.dot(q, w1b[...], preferred_element_type=jnp.float32)
        + b1[...])
    mlp = jax.nn.relu(
        jnp.dot(hidden, w2[...], preferred_element_type=jnp.float32)
        + b2[...])

    h_out[...] = h_new
    c_out[...] = c_new
    mlp_out[...] = mlp


def _finish_body(scores_ref, rel_ref, ent_ref, noise_ref,
                 logits_ref, aid_ref, loss_ref, chosen_ref, nexte_ref):
    rel = rel_ref[...]
    ent = ent_ref[...]
    noise = noise_ref[...]

    scores = jnp.where(rel == _PAD_ID, jnp.float32(-99999.0),
                       scores_ref[...])

    m = jnp.max(scores, axis=-1, keepdims=True)
    shifted = scores - m
    logits = shifted - jnp.log(jnp.sum(jnp.exp(shifted), axis=-1,
                                       keepdims=True))

    z = logits + noise
    zmax = jnp.max(z, axis=-1, keepdims=True)
    iota = lax.broadcasted_iota(jnp.int32, z.shape, 1)
    aid = jnp.min(jnp.where(z == zmax, iota, jnp.int32(_MAX_OUT)), axis=-1,
                  keepdims=True)

    sel = iota == aid
    loss = -jnp.sum(jnp.where(sel, logits, jnp.float32(0.0)), axis=-1,
                    keepdims=True)
    chosen = jnp.sum(jnp.where(sel, rel, jnp.int32(0)), axis=-1,
                     keepdims=True)
    nexte = jnp.sum(jnp.where(sel, ent, jnp.int32(0)), axis=-1,
                    keepdims=True)

    logits_ref[...] = logits
    aid_ref[...] = aid
    loss_ref[...] = loss
    chosen_ref[...] = chosen
    nexte_ref[...] = nexte


def kernel(prev_state_h, prev_state_c, prev_relation, current_entity,
           actions_id, queries, table, W_ih, W_hh, b_ih, b_hh, W1, b1, W2, b2):
    del current_entity

    rel2d = actions_id[:, :, 0]
    ent2d = actions_id[:, :, 1]

    tail_flat = table[_NBLK * 128:].reshape(-1)
    tab_lin = _sc_transpose(table.T, tail_flat).reshape(_V, _EMB)

    small_idx = jnp.concatenate(
        [prev_relation.astype(jnp.int32), queries.astype(jnp.int32)])
    small_rows = _sc_gather(tab_lin, small_idx, chunk=256)
    prev_emb = small_rows[:_B]
    q_emb = small_rows[_B:]

    wiht = W_ih.T
    whht = W_hh.T
    wx = [wiht[:, k * _STATE:(k + 1) * _STATE] for k in range(4)]
    wh = [whht[:, k * _STATE:(k + 1) * _STATE] for k in range(4)]
    bih4 = b_ih.reshape(4, _STATE)
    bhh4 = b_hh.reshape(4, _STATE)
    w1t = W1.T
    w1a = w1t[:_STATE]
    w1b = w1t[_STATE:]
    b1r = b1.reshape(1, _HID)
    w2t = W2.T
    b2r = b2.reshape(1, _EMB)

    bm = 512
    grid = _B // bm
    row_spec = pl.BlockSpec((bm, _EMB), lambda i: (i, 0))
    full = lambda shape: pl.BlockSpec(shape, lambda i: tuple(0 for _ in shape))
    h_new, c_new, mlp = pl.pallas_call(
        _dense_body,
        grid=(grid,),
        in_specs=[row_spec, row_spec, row_spec, row_spec]
        + [full((_EMB, _STATE))] * 8
        + [full((4, _STATE))] * 2
        + [full((_STATE, _HID)), full((_EMB, _HID)), full((1, _HID)),
           full((_HID, _EMB)), full((1, _EMB))],
        out_specs=[row_spec, row_spec, row_spec],
        out_shape=[jax.ShapeDtypeStruct((_B, _STATE), jnp.float32)] * 3,
    )(prev_emb, prev_state_h, prev_state_c, q_emb,
      *wx, *wh, bih4, bhh4, w1a, w1b, b1r, w2t, b2r)

    scores = _sc_scores(tab_lin, rel2d.reshape(-1), mlp)

    noise = jax.random.gumbel(jax.random.key(42), (_B, _MAX_OUT), jnp.float32)

    bm2 = 512
    grid2 = _B // bm2
    spec2d = pl.BlockSpec((bm2, _MAX_OUT), lambda i: (i, 0))
    spec1 = pl.BlockSpec((bm2, 1), lambda i: (i, 0))
    logits, aid, loss, chosen, nexte = pl.pallas_call(
        _finish_body,
        grid=(grid2,),
        in_specs=[spec2d, spec2d, spec2d, spec2d],
        out_specs=[spec2d, spec1, spec1, spec1, spec1],
        out_shape=[
            jax.ShapeDtypeStruct((_B, _MAX_OUT), jnp.float32),
            jax.ShapeDtypeStruct((_B, 1), jnp.int32),
            jax.ShapeDtypeStruct((_B, 1), jnp.float32),
            jax.ShapeDtypeStruct((_B, 1), jnp.int32),
            jax.ShapeDtypeStruct((_B, 1), jnp.int32),
        ],
    )(scores, rel2d, ent2d, noise)

    return (loss.reshape(_B), h_new, c_new, logits,
            aid.reshape(_B), nexte.reshape(_B), chosen.reshape(_B))
